# Initial kernel scaffold; baseline (speedup 1.0000x reference)
import jax
import jax.numpy as jnp
from jax.experimental import pallas as pl


def kernel(indices, table):
    raise NotImplementedError("write your pallas kernel here")


# speedup vs baseline: 1.1023x; 1.1023x over previous
import jax
import jax.numpy as jnp
from jax import lax
from jax.experimental import pallas as pl
from jax.experimental.pallas import tpu as pltpu
from jax.experimental.pallas import tpu_sc as plsc

NUM_EMBEDDINGS = 1000000
EMBED_DIM = 32
BATCH = 16384
TOKEN_LENGTH = 50

N = BATCH * TOKEN_LENGTH
NC, NS = 2, 16
NW = NC * NS
PER_W = N // NW
CH = 128
K = 8
GROUP = CH * K
G = PER_W // GROUP


def _body(idx_hbm, table_hbm, out_hbm, idx_v, rows_v, sem):
  wid = lax.axis_index("s") * NC + lax.axis_index("c")
  pltpu.sync_copy(idx_hbm.at[wid], idx_v)
  row_base = wid * PER_W

  def group(g, carry):
    for j in range(K):
      pltpu.async_copy(
          table_hbm.at[idx_v.at[g * K + j]],
          rows_v.at[pl.ds(j * CH, CH)],
          sem,
      )
    pltpu.make_async_copy(table_hbm.at[pl.ds(0, GROUP)], rows_v, sem).wait()
    pltpu.sync_copy(rows_v, out_hbm.at[pl.ds(row_base + g * GROUP, GROUP)])
    return carry

  lax.fori_loop(0, G, group, 0)


@jax.jit
def kernel(indices, table):
  idx = indices.reshape(NW, PER_W // CH, CH).astype(jnp.int32)
  call = pl.kernel(
      _body,
      out_type=jax.ShapeDtypeStruct((N, EMBED_DIM), jnp.float32),
      mesh=plsc.VectorSubcoreMesh(core_axis_name="c", subcore_axis_name="s"),
      scratch_types=[
          pltpu.VMEM((PER_W // CH, CH), jnp.int32),
          pltpu.VMEM((GROUP, EMBED_DIM), jnp.float32),
          pltpu.SemaphoreType.DMA,
      ],
      compiler_params=pltpu.CompilerParams(use_tc_tiling_on_sc=False),
  )
  out = call(idx, table)
  return out.reshape(BATCH, TOKEN_LENGTH, EMBED_DIM)


# speedup vs baseline: 1.1109x; 1.0077x over previous
import jax
import jax.numpy as jnp
from jax import lax
from jax.experimental import pallas as pl
from jax.experimental.pallas import tpu as pltpu
from jax.experimental.pallas import tpu_sc as plsc

NUM_EMBEDDINGS = 1000000
EMBED_DIM = 32
BATCH = 16384
TOKEN_LENGTH = 50

N = BATCH * TOKEN_LENGTH
NC, NS = 2, 16
NW = NC * NS
PER_W = N // NW
CH = 128
K = 10
GROUP = CH * K
G = PER_W // GROUP


def _body(idx_hbm, table_hbm, out_hbm, idx_v, rows_a, rows_b, sga, sgb,
          soa, sob):
  wid = lax.axis_index("s") * NC + lax.axis_index("c")
  pltpu.sync_copy(idx_hbm.at[wid], idx_v)
  row_base = wid * PER_W

  def fire(buf, sem, g):
    for j in range(K):
      pltpu.async_copy(
          table_hbm.at[idx_v.at[g * K + j]],
          buf.at[pl.ds(j * CH, CH)],
          sem,
      )

  def drain(buf, sem):
    pltpu.make_async_copy(table_hbm.at[pl.ds(0, GROUP)], buf, sem).wait()

  def out(buf, sem, g):
    pltpu.async_copy(buf, out_hbm.at[pl.ds(row_base + g * GROUP, GROUP)], sem)

  def wait_out(buf, sem):
    pltpu.make_async_copy(buf, out_hbm.at[pl.ds(0, GROUP)], sem).wait()

  def pair(k, carry):
    g = 2 * k

    @pl.when(k > 0)
    def _():
      wait_out(rows_b, sob)

    fire(rows_b, sgb, g + 1)
    drain(rows_a, sga)
    out(rows_a, soa, g)
    wait_out(rows_a, soa)

    @pl.when(k < G // 2 - 1)
    def _():
      fire(rows_a, sga, g + 2)

    drain(rows_b, sgb)
    out(rows_b, sob, g + 1)
    return carry

  fire(rows_a, sga, 0)
  lax.fori_loop(0, G // 2, pair, 0)
  wait_out(rows_b, sob)


@jax.jit
def kernel(indices, table):
  idx = indices.reshape(NW, PER_W // CH, CH).astype(jnp.int32)
  call = pl.kernel(
      _body,
      out_type=jax.ShapeDtypeStruct((N, EMBED_DIM), jnp.float32),
      mesh=plsc.VectorSubcoreMesh(core_axis_name="c", subcore_axis_name="s"),
      scratch_types=[
          pltpu.VMEM((PER_W // CH, CH), jnp.int32),
          pltpu.VMEM((GROUP, EMBED_DIM), jnp.float32),
          pltpu.VMEM((GROUP, EMBED_DIM), jnp.float32),
          pltpu.SemaphoreType.DMA,
          pltpu.SemaphoreType.DMA,
          pltpu.SemaphoreType.DMA,
          pltpu.SemaphoreType.DMA,
      ],
      compiler_params=pltpu.CompilerParams(use_tc_tiling_on_sc=False),
  )
  out = call(idx, table)
  return out.reshape(BATCH, TOKEN_LENGTH, EMBED_DIM)


# speedup vs baseline: 1.8055x; 1.6253x over previous
import jax
import jax.numpy as jnp
from jax import lax
from jax.experimental import pallas as pl
from jax.experimental.pallas import tpu as pltpu
from jax.experimental.pallas import tpu_sc as plsc

NUM_EMBEDDINGS = 1000000
EMBED_DIM = 32
BATCH = 16384
TOKEN_LENGTH = 50

NC, NS = 2, 16
NW = NC * NS
B_PER_W = BATCH // NW
GB = 16
G = B_PER_W // GB


def _body(idx_hbm, table_hbm, out_hbm, idx_v, rows_a, rows_b, sga, sgb,
          soa, sob):
  wid = lax.axis_index("s") * NC + lax.axis_index("c")
  pltpu.sync_copy(idx_hbm.at[wid], idx_v)
  b_base = wid * B_PER_W

  def fire(buf, sem, g):
    for j in range(GB):
      pltpu.async_copy(
          table_hbm.at[idx_v.at[g * GB + j]],
          buf.at[j],
          sem,
      )

  def drain(buf, sem):
    pltpu.make_async_copy(out_hbm.at[pl.ds(0, GB)], buf, sem).wait()

  def out(buf, sem, g):
    pltpu.async_copy(buf, out_hbm.at[pl.ds(b_base + g * GB, GB)], sem)

  def wait_out(buf, sem):
    pltpu.make_async_copy(buf, out_hbm.at[pl.ds(0, GB)], sem).wait()

  def pair(k, carry):
    g = 2 * k

    @pl.when(k > 0)
    def _():
      wait_out(rows_b, sob)

    fire(rows_b, sgb, g + 1)
    drain(rows_a, sga)
    out(rows_a, soa, g)
    wait_out(rows_a, soa)

    @pl.when(k < G // 2 - 1)
    def _():
      fire(rows_a, sga, g + 2)

    drain(rows_b, sgb)
    out(rows_b, sob, g + 1)
    return carry

  fire(rows_a, sga, 0)
  lax.fori_loop(0, G // 2, pair, 0)
  wait_out(rows_b, sob)


@jax.jit
def kernel(indices, table):
  idx = indices.reshape(NW, B_PER_W, TOKEN_LENGTH).astype(jnp.int32)
  call = pl.kernel(
      _body,
      out_type=jax.ShapeDtypeStruct((BATCH, TOKEN_LENGTH, EMBED_DIM),
                                    jnp.float32),
      mesh=plsc.VectorSubcoreMesh(core_axis_name="c", subcore_axis_name="s"),
      scratch_types=[
          pltpu.VMEM((B_PER_W, TOKEN_LENGTH), jnp.int32),
          pltpu.VMEM((GB, TOKEN_LENGTH, EMBED_DIM), jnp.float32),
          pltpu.VMEM((GB, TOKEN_LENGTH, EMBED_DIM), jnp.float32),
          pltpu.SemaphoreType.DMA,
          pltpu.SemaphoreType.DMA,
          pltpu.SemaphoreType.DMA,
          pltpu.SemaphoreType.DMA,
      ],
      compiler_params=pltpu.CompilerParams(use_tc_tiling_on_sc=False),
  )
  return call(idx, table)
